# baseline (speedup 1.0000x reference)
import functools

import jax
import jax.numpy as jnp
from jax import lax
from jax.experimental import pallas as pl
from jax.experimental.pallas import tpu as pltpu
from jax.experimental.pallas import tpu_sc as plsc

_NB = 16
_ALPHA = 0.1
_LEVELS = float((1 << _NB) - 1)
_CHUNK = 2048


@functools.lru_cache(maxsize=None)
def _make_sc_kernel(n_elems: int):
    info = plsc.get_sparse_core_info()
    nc, ns, L = info.num_cores, info.num_subcores, info.num_lanes
    nw = nc * ns
    per_w = n_elems // nw
    n_chunks = per_w // _CHUNK
    mesh = plsc.VectorSubcoreMesh(core_axis_name="c", subcore_axis_name="s")

    @functools.partial(
        pl.kernel,
        mesh=mesh,
        out_type=jax.ShapeDtypeStruct((n_elems * _NB,), jnp.float32),
        scratch_types=[
            pltpu.VMEM((_CHUNK,), jnp.float32),
            pltpu.VMEM((_CHUNK * _NB,), jnp.float32),
        ],
        compiler_params=pltpu.CompilerParams(needs_layout_passes=False),
    )
    def gc_kernel(x_hbm, out_hbm, x_v, out_v):
        wid = lax.axis_index("s") * nc + lax.axis_index("c")
        lane16 = jax.lax.iota(jnp.int32, L) * _NB
        zeros = jnp.zeros((L,), jnp.float32)

        def chunk_body(ci, carry):
            base = wid * per_w + ci * _CHUNK
            pltpu.sync_copy(x_hbm.at[pl.ds(base, _CHUNK)], x_v)

            def v_body(v, carry2):
                xv = x_v[pl.ds(v * L, L)]
                q = jnp.clip((xv * _LEVELS).astype(jnp.int32), 0, (1 << _NB) - 1)
                g = jnp.bitwise_xor(q, jnp.right_shift(q, 1))
                scale = xv * _ALPHA + 1.0
                idxv = lane16 + v * (L * _NB)
                for i in range(_NB):
                    m = jnp.bitwise_and(g, (1 << i)) != 0
                    val = jnp.where(m, scale, zeros)
                    plsc.store_scatter(out_v, [idxv + i], val)
                return carry2

            lax.fori_loop(0, _CHUNK // L, v_body, 0)
            pltpu.sync_copy(out_v, out_hbm.at[pl.ds(base * _NB, _CHUNK * _NB)])
            return carry

        lax.fori_loop(0, n_chunks, chunk_body, 0)

    return gc_kernel


def kernel(x):
    n = x.shape[0] * x.shape[1]
    out_flat = _make_sc_kernel(n)(x.reshape((n,)))
    return out_flat.reshape(x.shape[:-1] + (_NB,))


# speedup vs baseline: 6.3745x; 6.3745x over previous
import functools

import jax
import jax.numpy as jnp
from jax import lax
from jax.experimental import pallas as pl
from jax.experimental.pallas import tpu as pltpu
from jax.experimental.pallas import tpu_sc as plsc

_NB = 16
_ALPHA = 0.1
_LEVELS = float((1 << _NB) - 1)
_D0 = 16384
_D1 = 200


@functools.lru_cache(maxsize=None)
def _make_sc_kernel():
    info = plsc.get_sparse_core_info()
    nc, ns, L = info.num_cores, info.num_subcores, info.num_lanes
    nw = nc * ns
    d1_half = _D1 // 2
    bgrp = _D0 // 128 // (nw // 2)
    mesh = plsc.VectorSubcoreMesh(core_axis_name="c", subcore_axis_name="s")

    @functools.partial(
        pl.kernel,
        mesh=mesh,
        out_type=jax.ShapeDtypeStruct((_D1 * _D0 * _NB // 128, 128), jnp.float32),
        scratch_types=[
            pltpu.VMEM((bgrp * 128,), jnp.float32),
            pltpu.VMEM((2 * bgrp * 8, 128), jnp.float32),
        ],
        compiler_params=pltpu.CompilerParams(
            needs_layout_passes=False, use_tc_tiling_on_sc=True
        ),
    )
    def gc_kernel(x_hbm, out_hbm, x_v, out_v):
        wid = lax.axis_index("s") * nc + lax.axis_index("c")
        half = wid // (nw // 2)
        bq = wid % (nw // 2)
        d1_base = half * d1_half

        def d1_body(j, carry):
            d1 = d1_base + j
            pltpu.sync_copy(
                x_hbm.at[pl.ds(d1 * _D0 + bq * (bgrp * 128), bgrp * 128)], x_v
            )

            def blk(b, carry2):
                xv = x_v[pl.ds(b * L, L)]
                q = jnp.clip((xv * _LEVELS).astype(jnp.int32), 0, (1 << _NB) - 1)
                g = jnp.bitwise_xor(q, jnp.right_shift(q, 1))
                scale = xv * _ALPHA + 1.0
                zeros = jnp.zeros((L,), jnp.float32)
                for i in range(_NB):
                    m = jnp.bitwise_and(g, (1 << i)) != 0
                    val = jnp.where(m, scale, zeros)
                    row = (i // 8) * (bgrp * 8) + (b // 8) * 8 + (i % 8)
                    out_v[row, pl.ds((b % 8) * L, L)] = val
                return carry2

            lax.fori_loop(0, bgrp * 8, blk, 0)
            rows_per_d1 = _D0 * _NB // 128
            half_rows = rows_per_d1 // 2
            run = bgrp * 8
            row0 = d1 * rows_per_d1 + bq * run
            pltpu.sync_copy(out_v.at[pl.ds(0, run)], out_hbm.at[pl.ds(row0, run)])
            pltpu.sync_copy(
                out_v.at[pl.ds(run, run)], out_hbm.at[pl.ds(row0 + half_rows, run)]
            )
            return carry

        lax.fori_loop(0, d1_half, d1_body, 0)

    return gc_kernel


def kernel(x):
    x_lin = jnp.transpose(x.reshape(_D0, _D1)).reshape(_D0 * _D1)
    out_lin = _make_sc_kernel()(x_lin)
    out6 = out_lin.reshape(_D1, 2, _D0 // 128, 8, 128)
    return jnp.transpose(out6, (2, 4, 0, 1, 3)).reshape(_D0, _D1, _NB)


# speedup vs baseline: 11.7928x; 1.8500x over previous
import functools

import jax
import jax.numpy as jnp
from jax import lax
from jax.experimental import pallas as pl
from jax.experimental.pallas import tpu as pltpu
from jax.experimental.pallas import tpu_sc as plsc

_NB = 16
_ALPHA = 0.1
_LEVELS = float((1 << _NB) - 1)
_D0 = 16384
_D1 = 200


@functools.lru_cache(maxsize=None)
def _make_sc_kernel():
    info = plsc.get_sparse_core_info()
    nc, ns, L = info.num_cores, info.num_subcores, info.num_lanes
    nw = nc * ns
    d1_half = _D1 // 2
    cols = _D0 // (nw // 2)
    run = cols // L // 8 * 8
    rows_per_d1 = _D0 * _NB // 128
    half_rows = rows_per_d1 // 2
    mesh = plsc.VectorSubcoreMesh(core_axis_name="c", subcore_axis_name="s")

    @functools.partial(
        pl.kernel,
        mesh=mesh,
        out_type=jax.ShapeDtypeStruct((_D1 * _D0 * _NB // 128, 128), jnp.float32),
        scratch_types=[
            pltpu.VMEM((2, cols), jnp.float32),
            pltpu.VMEM((2, 2 * run, 128), jnp.float32),
            pltpu.SemaphoreType.DMA,
            pltpu.SemaphoreType.DMA,
            pltpu.SemaphoreType.DMA,
            pltpu.SemaphoreType.DMA,
        ],
        compiler_params=pltpu.CompilerParams(
            needs_layout_passes=False, use_tc_tiling_on_sc=True
        ),
    )
    def gc_kernel(x_hbm, out_hbm, x_v, out_v, si0, si1, so0, so1):
        wid = lax.axis_index("s") * nc + lax.axis_index("c")
        half = wid // (nw // 2)
        bq = wid % (nw // 2)
        d1_base = half * d1_half
        sems_in = (si0, si1)
        sems_out = (so0, so1)

        def x_src(j):
            return x_hbm.at[pl.ds((d1_base + j) * _D0 + bq * cols, cols)]

        pltpu.async_copy(x_src(0), x_v.at[0], sems_in[0])
        pltpu.async_copy(x_src(1), x_v.at[1], sems_in[1])

        def iter_body(p, j):
            d1 = d1_base + j
            row0 = d1 * rows_per_d1 + bq * run
            pltpu.make_async_copy(x_src(j), x_v.at[p], sems_in[p]).wait()

            @pl.when(j >= 2)
            def _wait_out():
                pltpu.make_async_copy(
                    out_v.at[p, pl.ds(0, run)], out_hbm.at[pl.ds(row0, run)],
                    sems_out[p],
                ).wait()
                pltpu.make_async_copy(
                    out_v.at[p, pl.ds(run, run)],
                    out_hbm.at[pl.ds(row0 + half_rows, run)], sems_out[p],
                ).wait()

            def blk(b, carry2):
                xv = x_v[p, pl.ds(b * L, L)]
                q = (xv * _LEVELS).astype(jnp.int32)
                g = jnp.bitwise_xor(q, jnp.right_shift(q, 1))
                scale = xv * _ALPHA + 1.0
                zeros = jnp.zeros((L,), jnp.float32)
                for i in range(_NB):
                    m = jnp.bitwise_and(g, (1 << i)) != 0
                    val = jnp.where(m, scale, zeros)
                    row = (i // 8) * run + (b // 8) * 8 + (i % 8)
                    out_v[p, row, pl.ds((b % 8) * L, L)] = val
                return carry2

            lax.fori_loop(0, cols // L, blk, 0, unroll=2)

            @pl.when(j + 2 < d1_half)
            def _prefetch():
                pltpu.async_copy(x_src(j + 2), x_v.at[p], sems_in[p])

            pltpu.async_copy(
                out_v.at[p, pl.ds(0, run)], out_hbm.at[pl.ds(row0, run)],
                sems_out[p],
            )
            pltpu.async_copy(
                out_v.at[p, pl.ds(run, run)],
                out_hbm.at[pl.ds(row0 + half_rows, run)], sems_out[p],
            )

        def outer(jj, carry):
            iter_body(0, jj * 2)
            iter_body(1, jj * 2 + 1)
            return carry

        lax.fori_loop(0, d1_half // 2, outer, 0)

        for p in range(2):
            d1 = d1_base + d1_half - 2 + p
            row0 = d1 * rows_per_d1 + bq * run
            pltpu.make_async_copy(
                out_v.at[p, pl.ds(0, run)], out_hbm.at[pl.ds(row0, run)],
                sems_out[p],
            ).wait()
            pltpu.make_async_copy(
                out_v.at[p, pl.ds(run, run)],
                out_hbm.at[pl.ds(row0 + half_rows, run)], sems_out[p],
            ).wait()

    return gc_kernel


def kernel(x):
    x_lin = jnp.transpose(x.reshape(_D0, _D1)).reshape(_D0 * _D1)
    out_lin = _make_sc_kernel()(x_lin)
    out6 = out_lin.reshape(_D1, 2, _D0 // 128, 8, 128)
    return jnp.transpose(out6, (2, 4, 0, 1, 3)).reshape(_D0, _D1, _NB)
